# baseline (speedup 1.0000x reference)
import functools

import jax
import jax.numpy as jnp
from jax import lax
from jax.experimental import pallas as pl
from jax.experimental.pallas import tpu as pltpu
from jax.experimental.pallas import tpu_sc as plsc

_B = 16384
_L = 50
_D = 16
_ST = 5
_OUT = 64

_NC, _NS = 2, 16
_NW = _NC * _NS
_RPW = _B // _NW
_CH = 64
_NCHUNK = _RPW // _CH
_GI = _CH * _L


def _sc_pool(ids, table, zrs, seg):
    mesh = plsc.VectorSubcoreMesh(core_axis_name="c", subcore_axis_name="s")

    @functools.partial(
        pl.kernel,
        mesh=mesh,
        compiler_params=pltpu.CompilerParams(use_tc_tiling_on_sc=False),
        out_type=jax.ShapeDtypeStruct((_B, _D), jnp.float32),
        scratch_types=[
            pltpu.VMEM((_GI,), jnp.int32),
            pltpu.VMEM((_GI,), jnp.int32),
            pltpu.VMEM((_GI,), jnp.int32),
            pltpu.VMEM((_GI, _D), jnp.float32),
            pltpu.VMEM((_GI, _D), jnp.float32),
            pltpu.VMEM((_CH, _D), jnp.float32),
            pltpu.SemaphoreType.DMA,
            pltpu.SemaphoreType.DMA,
            pltpu.VMEM_SHARED((_NS * _CH, _D), jnp.float32),
        ],
    )
    def body(ids_hbm, table_hbm, z_hbm, seg_hbm, out_hbm,
             idx_a, idx_b, seg_v, rows_a, rows_b, z_v,
             gsem_a, gsem_b, pool_s):
        sid = lax.axis_index("s")
        wid = sid * _NC + lax.axis_index("c")
        pltpu.sync_copy(seg_hbm.at[pl.ds(sid * _GI, _GI)], seg_v)
        pltpu.sync_copy(z_hbm, z_v)
        bufs = ((idx_a, rows_a, gsem_a), (idx_b, rows_b, gsem_b))

        def load_and_gather(c):
            idx, rows, gsem = bufs[c % 2]
            base = wid * _RPW + c * _CH
            pltpu.sync_copy(ids_hbm.at[pl.ds(base * _L, _GI)], idx)
            return pltpu.async_copy(table_hbm.at[idx], rows, gsem)

        gathers = [load_and_gather(0)] + [None] * (_NCHUNK - 1)
        for c in range(_NCHUNK):
            _, rows, _ = bufs[c % 2]
            if c + 1 < _NCHUNK:
                gathers[c + 1] = load_and_gather(c + 1)
            gathers[c].wait()
            base = wid * _RPW + c * _CH
            pltpu.sync_copy(z_v, pool_s.at[pl.ds(sid * _CH, _CH)])
            pltpu.sync_copy(rows, pool_s.at[seg_v], add=True)
            pltpu.sync_copy(pool_s.at[pl.ds(sid * _CH, _CH)],
                            out_hbm.at[pl.ds(base, _CH)])

    return body(ids, table, zrs, seg)


def _tc_head(ids, sums, stats, w, b):
    rows = 2048
    grid = (_B // rows,)

    def body(ids_ref, sums_ref, stats_ref, w_ref, b_ref, out_ref):
        idv = ids_ref[...]
        cnt = jnp.sum((idv != 0).astype(jnp.float32), axis=1)
        inv = 1.0 / jnp.maximum(cnt, 1.0)
        wm = w_ref[...]
        accp = lax.dot_general(wm[:, :_D], sums_ref[...],
                               (((1,), (1,)), ((), ())),
                               preferred_element_type=jnp.float32)
        accs = lax.dot_general(wm[:, _D:], stats_ref[...],
                               (((1,), (1,)), ((), ())),
                               preferred_element_type=jnp.float32)
        acc = accp * inv[None, :] + accs + b_ref[...]
        out_ref[...] = jnp.maximum(acc, 0.0)

    return pl.pallas_call(
        body,
        grid=grid,
        in_specs=[
            pl.BlockSpec((rows, _L), lambda i: (i, 0)),
            pl.BlockSpec((rows, _D), lambda i: (i, 0)),
            pl.BlockSpec((rows, _ST), lambda i: (i, 0)),
            pl.BlockSpec((_OUT, _D + _ST), lambda i: (0, 0)),
            pl.BlockSpec((_OUT, 1), lambda i: (0, 0)),
        ],
        out_specs=pl.BlockSpec((_OUT, rows), lambda i: (0, i)),
        out_shape=jax.ShapeDtypeStruct((_OUT, _B), jnp.float32),
    )(ids, sums, stats, w, b.reshape(_OUT, 1))


def kernel(skill_ids, stats, skill_emb, proj_W, proj_b):
    ids8 = (skill_ids * 8).reshape(_B * _L)
    tbl8 = jnp.pad(skill_emb, ((0, 7), (0, 112))).reshape(8 * 100008, _D)
    seg = (jnp.arange(_GI, dtype=jnp.int32) // _L)[None, :] + (
        jnp.arange(_NS, dtype=jnp.int32) * _CH)[:, None]
    seg = seg.reshape(_NS * _GI)
    zrs = jnp.zeros((_CH, _D), jnp.float32)
    sums = _sc_pool(ids8, tbl8, zrs, seg)
    out_t = _tc_head(skill_ids, sums, stats, proj_W, proj_b)
    return out_t.T


# speedup vs baseline: 1.2352x; 1.2352x over previous
import functools

import jax
import jax.numpy as jnp
from jax import lax
from jax.experimental import pallas as pl
from jax.experimental.pallas import tpu as pltpu
from jax.experimental.pallas import tpu_sc as plsc

_B = 16384
_L = 50
_D = 16
_ST = 5
_OUT = 64

_NC, _NS = 2, 16
_NW = _NC * _NS
_RPW = _B // _NW
_CH = 64
_NCHUNK = _RPW // _CH
_GI = _CH * _L


_TR = 100352
_SPAN = _TR // _NW
_NG = _SPAN // 16


def _sc_transpose(table_t):
    mesh = plsc.VectorSubcoreMesh(core_axis_name="c", subcore_axis_name="s")

    @functools.partial(
        pl.kernel,
        mesh=mesh,
        compiler_params=pltpu.CompilerParams(
            use_tc_tiling_on_sc=False, needs_layout_passes=False),
        out_type=jax.ShapeDtypeStruct((_TR * _D,), jnp.float32),
        scratch_types=[
            pltpu.VMEM((_D * _SPAN,), jnp.float32),
            pltpu.VMEM((_SPAN * _D,), jnp.float32),
            pltpu.SemaphoreType.DMA,
        ],
    )
    def body(tt_hbm, out_hbm, in_v, out_v, sem):
        wid = lax.axis_index("s") * _NC + lax.axis_index("c")
        start = wid * _SPAN
        copies = [
            pltpu.async_copy(tt_hbm.at[d, pl.ds(start, _SPAN)],
                             in_v.at[pl.ds(d * _SPAN, _SPAN)], sem)
            for d in range(_D)
        ]
        for cp in copies:
            cp.wait()
        lanes = lax.iota(jnp.int32, 16) * _D

        @pl.loop(0, _NG)
        def _(g):
            gbase = g * (16 * _D)
            for d in range(_D):
                v = in_v[pl.ds(d * _SPAN + g * 16, 16)]
                plsc.store_scatter(out_v, [lanes + (gbase + d)], v)

        pltpu.sync_copy(out_v, out_hbm.at[pl.ds(start * _D, _SPAN * _D)])

    return body(table_t)


def _sc_pool(ids, table):
    mesh = plsc.VectorSubcoreMesh(core_axis_name="c", subcore_axis_name="s")

    @functools.partial(
        pl.kernel,
        mesh=mesh,
        compiler_params=pltpu.CompilerParams(
            use_tc_tiling_on_sc=False, needs_layout_passes=False),
        out_type=jax.ShapeDtypeStruct((_B * _D,), jnp.float32),
        scratch_types=[
            pltpu.VMEM((_GI,), jnp.int32),
            pltpu.VMEM((_GI,), jnp.int32),
            pltpu.VMEM((_GI, _D), jnp.float32),
            pltpu.VMEM((_GI, _D), jnp.float32),
            pltpu.VMEM((_CH * _D,), jnp.float32),
            pltpu.VMEM((_CH * _D,), jnp.float32),
            pltpu.VMEM((_CH,), jnp.float32),
            pltpu.SemaphoreType.DMA,
            pltpu.SemaphoreType.DMA,
            pltpu.SemaphoreType.DMA,
            pltpu.SemaphoreType.DMA,
        ],
    )
    def body(ids_hbm, table_hbm, out_hbm,
             idx_a, idx_b, rows_a, rows_b, sums_a, sums_b, inv_v,
             gsem_a, gsem_b, osem_a, osem_b):
        wid = lax.axis_index("s") * _NC + lax.axis_index("c")
        iota16 = lax.iota(jnp.int32, 16)
        bufs = ((idx_a, rows_a, sums_a, gsem_a, osem_a),
                (idx_b, rows_b, sums_b, gsem_b, osem_b))

        def load_and_gather(c):
            idx, rows, _, gsem, _ = bufs[c % 2]
            base = wid * _RPW + c * _CH
            pltpu.sync_copy(ids_hbm.at[pl.ds(base * _L, _GI)], idx)
            return pltpu.async_copy(table_hbm.at[idx], rows, gsem)

        gathers = [load_and_gather(0)] + [None] * (_NCHUNK - 1)
        outs = [None] * _NCHUNK
        for c in range(_NCHUNK):
            idx, rows, sums_v, _, osem = bufs[c % 2]
            if c + 1 < _NCHUNK:
                gathers[c + 1] = load_and_gather(c + 1)
            if c >= 2:
                outs[c - 2].wait()

            @pl.loop(0, _CH // 16)
            def _(q):
                pos = iota16 * _L + q * (16 * _L)
                cnts = [jnp.zeros((16,), jnp.float32) for _ in range(4)]
                for j in range(_L):
                    v = plsc.load_gather(idx, [pos + j])
                    cnts[j % 4] = cnts[j % 4] + (v != 0).astype(jnp.float32)
                cnt = (cnts[0] + cnts[1]) + (cnts[2] + cnts[3])
                inv_v[pl.ds(q * 16, 16)] = 1.0 / jnp.maximum(cnt, 1.0)

            gathers[c].wait()

            @pl.loop(0, _CH)
            def _(p):
                accs = [rows[p * _L + j, :] for j in range(4)]
                for j in range(4, _L):
                    accs[j % 4] = accs[j % 4] + rows[p * _L + j, :]
                acc = (accs[0] + accs[1]) + (accs[2] + accs[3])
                iv = plsc.load_gather(inv_v, [jnp.broadcast_to(p, (16,))])
                sums_v[pl.ds(p * _D, _D)] = acc * iv

            base = wid * _RPW + c * _CH
            outs[c] = pltpu.async_copy(
                sums_v, out_hbm.at[pl.ds(base * _D, _CH * _D)], osem)
        outs[_NCHUNK - 2].wait()
        outs[_NCHUNK - 1].wait()

    return body(ids, table)


def _tc_head(pooled, stats, w, b):
    rows = 2048
    grid = (_B // rows,)

    def body(pooled_ref, stats_ref, w_ref, b_ref, out_ref):
        wm = w_ref[...]
        accp = lax.dot_general(wm[:, :_D], pooled_ref[...],
                               (((1,), (1,)), ((), ())),
                               preferred_element_type=jnp.float32)
        accs = lax.dot_general(wm[:, _D:], stats_ref[...],
                               (((1,), (1,)), ((), ())),
                               preferred_element_type=jnp.float32)
        out_ref[...] = jnp.maximum(accp + accs + b_ref[...], 0.0)

    return pl.pallas_call(
        body,
        grid=grid,
        in_specs=[
            pl.BlockSpec((rows, _D), lambda i: (i, 0)),
            pl.BlockSpec((rows, _ST), lambda i: (i, 0)),
            pl.BlockSpec((_OUT, _D + _ST), lambda i: (0, 0)),
            pl.BlockSpec((_OUT, 1), lambda i: (0, 0)),
        ],
        out_specs=pl.BlockSpec((_OUT, rows), lambda i: (0, i)),
        out_shape=jax.ShapeDtypeStruct((_OUT, _B), jnp.float32),
    )(pooled, stats, w, b.reshape(_OUT, 1))


def kernel(skill_ids, stats, skill_emb, proj_W, proj_b):
    ids_flat = skill_ids.reshape(_B * _L)
    table_t = jnp.pad(skill_emb.T, ((0, 0), (0, _TR - 100001)))
    tbl_lin = _sc_transpose(table_t).reshape(_TR, _D)
    pooled = _sc_pool(ids_flat, tbl_lin).reshape(_B, _D)
    out_t = _tc_head(pooled, stats, proj_W, proj_b)
    return out_t.T
